# baseline (speedup 1.0000x reference)
import functools

import jax
import jax.numpy as jnp
from jax.experimental import pallas as pl
from jax.experimental.pallas import tpu as pltpu

DIM = 512
INTER = 1344
TOPK = 2
E = 64
NSHARE = 2
ROUTE_SCALE = 1.0
ALPHA = 0.1
N = 2048

T = 64
NT = 128
SC = 1024
NC = (NT * T) // SC


def _silu(v):
    return v * jax.nn.sigmoid(v)


def _router_body(x_ref, wg_ref, pairs_ref, cnt_ref, aux_ref):
    x = x_ref[...]
    wg = wg_ref[...]
    logits = jax.lax.dot_general(
        x, wg, (((1,), (1,)), ((), ())), preferred_element_type=jnp.float32)
    m = jnp.max(logits, axis=1, keepdims=True)
    ex = jnp.exp(logits - m)
    score = ex / jnp.sum(ex, axis=1, keepdims=True)

    iota = jax.lax.broadcasted_iota(jnp.int32, (N, E), 1)
    m1 = jnp.max(score, axis=1, keepdims=True)
    i1 = jnp.min(jnp.where(score == m1, iota, E), axis=1, keepdims=True)
    sc2 = jnp.where(iota == i1, -1.0, score)
    m2 = jnp.max(sc2, axis=1, keepdims=True)
    i2 = jnp.min(jnp.where(sc2 == m2, iota, E), axis=1, keepdims=True)

    oh1 = (iota == i1).astype(jnp.float32)
    oh2 = (iota == i2).astype(jnp.float32)
    oh12 = oh1 + oh2

    cnt = jnp.sum(oh12, axis=0, keepdims=True)
    fi = cnt / (TOPK * N / E)
    pi = jnp.mean(score, axis=0, keepdims=True)
    aux = jnp.sum(fi * pi) * ALPHA
    aux_ref[...] = jnp.full((8, 128), aux, jnp.float32)

    ir = jax.lax.broadcasted_iota(jnp.int32, (N, N), 0)
    ic = jax.lax.broadcasted_iota(jnp.int32, (N, N), 1)
    ltri = (ic < ir).astype(jnp.float32)
    cum = jax.lax.dot_general(ltri, oh12, (((1,), (0,)), ((), ())),
                              preferred_element_type=jnp.float32)
    rank1 = jnp.sum(cum * oh1, axis=1, keepdims=True)
    rank2 = jnp.sum(cum * oh2, axis=1, keepdims=True)

    tiles_per = jnp.floor((cnt + (T - 1)) * (1.0 / T))
    er = jax.lax.broadcasted_iota(jnp.int32, (E, E), 0)
    ec = jax.lax.broadcasted_iota(jnp.int32, (E, E), 1)
    etri = (er < ec).astype(jnp.float32)
    tile_start = jax.lax.dot_general(
        tiles_per, etri, (((1,), (0,)), ((), ())),
        preferred_element_type=jnp.float32)
    base1 = jnp.sum(tile_start * oh1, axis=1, keepdims=True) * T
    base2 = jnp.sum(tile_start * oh2, axis=1, keepdims=True) * T
    s1 = base1 + rank1
    s2 = base2 + rank2

    zf = jnp.zeros((N, 4), jnp.float32)
    pairs_ref[...] = jnp.concatenate([s1, s2, m1, m2, zf], axis=1)
    cnt_ref[...] = jnp.concatenate(
        [jnp.concatenate([cnt, jnp.zeros((1, 128 - E), jnp.float32)], axis=1),
         jnp.zeros((7, 128), jnp.float32)], axis=0)


def _grouped_body(te_ref, tv_ref, x_ref, srow_ref, w1a_ref, w1b_ref,
                  w2a_ref, w2b_ref, w3a_ref, w3b_ref, og_ref):
    t = pl.program_id(0)

    @pl.when(tv_ref[t] == 1)
    def _live():
        s1r = srow_ref[0:1, :].astype(jnp.int32)
        s2r = srow_ref[1:2, :].astype(jnp.int32)
        jglob = jax.lax.broadcasted_iota(jnp.int32, (T, N), 0) + t * T
        onehot_g = ((jglob == s1r) | (jglob == s2r)).astype(jnp.float32)
        xs = jax.lax.dot_general(
            onehot_g, x_ref[...], (((1,), (0,)), ((), ())),
            preferred_element_type=jnp.float32)

        def half(w1_ref, w3_ref, w2_ref):
            w1 = w1_ref[0]
            w3 = w3_ref[0]
            w2t = w2_ref[0]
            h1 = jax.lax.dot_general(xs, w1, (((1,), (1,)), ((), ())),
                                     preferred_element_type=jnp.float32)
            h3 = jax.lax.dot_general(xs, w3, (((1,), (1,)), ((), ())),
                                     preferred_element_type=jnp.float32)
            h = _silu(h1) * h3
            return jax.lax.dot_general(
                h, w2t, (((1,), (0,)), ((), ())),
                preferred_element_type=jnp.float32)

        og_ref[...] = (half(w1a_ref, w3a_ref, w2a_ref)
                       + half(w1b_ref, w3b_ref, w2b_ref))

    @pl.when(tv_ref[t] == 0)
    def _dead():
        og_ref[...] = jnp.zeros_like(og_ref)


def _scatter_body(cv_ref, pairs_ref, og_ref, z_ref, y_ref):
    c = pl.program_id(0)

    @pl.when(c == 0)
    def _init():
        y_ref[...] = z_ref[...]

    @pl.when(cv_ref[c] == 1)
    def _live():
        s1c = pairs_ref[:, 0:1].astype(jnp.int32)
        s2c = pairs_ref[:, 1:2].astype(jnp.int32)
        g1c = pairs_ref[:, 2:3]
        g2c = pairs_ref[:, 3:4]
        jglob = jax.lax.broadcasted_iota(jnp.int32, (N, SC), 1) + c * SC
        scat = ((jglob == s1c).astype(jnp.float32) * g1c
                + (jglob == s2c).astype(jnp.float32) * g2c)
        y_ref[...] += jax.lax.dot_general(
            scat, og_ref[...], (((1,), (0,)), ((), ())),
            preferred_element_type=jnp.float32)


def _shared_body(x_ref, w1_ref, w2_ref, w3_ref, o_ref):
    xs = x_ref[...]
    h1 = jax.lax.dot_general(xs, w1_ref[...], (((1,), (1,)), ((), ())),
                             preferred_element_type=jnp.float32)
    h3 = jax.lax.dot_general(xs, w3_ref[...], (((1,), (1,)), ((), ())),
                             preferred_element_type=jnp.float32)
    h = _silu(h1) * h3
    o_ref[...] = jax.lax.dot_general(h, w2_ref[...], (((1,), (1,)), ((), ())),
                                     preferred_element_type=jnp.float32)


@functools.partial(jax.jit, static_argnames=())
def kernel(x, Wg, W1, W2, W3, Ws1, Ws2, Ws3):
    b, n, d = x.shape
    xf = x.reshape(n, d)

    pairs, cnt8, aux8 = pl.pallas_call(
        _router_body,
        out_shape=[
            jax.ShapeDtypeStruct((N, 8), jnp.float32),
            jax.ShapeDtypeStruct((8, 128), jnp.float32),
            jax.ShapeDtypeStruct((8, 128), jnp.float32),
        ],
    )(xf, Wg)
    aux_loss = aux8[0, 0]

    counts = cnt8[0, :E].astype(jnp.int32)
    tiles_per = (counts + T - 1) // T
    bounds = jnp.cumsum(tiles_per)
    total_tiles = bounds[-1]
    tile_expert = jnp.minimum(
        jnp.searchsorted(bounds, jnp.arange(NT, dtype=jnp.int32),
                         side='right'),
        E - 1).astype(jnp.int32)
    tile_valid = (jnp.arange(NT, dtype=jnp.int32)
                  < total_tiles).astype(jnp.int32)
    chunk_valid = (jnp.arange(NC, dtype=jnp.int32) * SC
                   < total_tiles * T).astype(jnp.int32)
    srow = pairs.T

    og = pl.pallas_call(
        _grouped_body,
        grid_spec=pltpu.PrefetchScalarGridSpec(
            num_scalar_prefetch=2,
            grid=(NT,),
            in_specs=[
                pl.BlockSpec((N, DIM), lambda t, te, tv: (0, 0)),
                pl.BlockSpec((8, N), lambda t, te, tv: (0, 0)),
                pl.BlockSpec((1, INTER // 2, DIM),
                             lambda t, te, tv: (te[t], 0, 0)),
                pl.BlockSpec((1, INTER // 2, DIM),
                             lambda t, te, tv: (te[t], 1, 0)),
                pl.BlockSpec((1, INTER // 2, DIM),
                             lambda t, te, tv: (te[t], 0, 0)),
                pl.BlockSpec((1, INTER // 2, DIM),
                             lambda t, te, tv: (te[t], 1, 0)),
                pl.BlockSpec((1, INTER // 2, DIM),
                             lambda t, te, tv: (te[t], 0, 0)),
                pl.BlockSpec((1, INTER // 2, DIM),
                             lambda t, te, tv: (te[t], 1, 0)),
            ],
            out_specs=pl.BlockSpec((T, DIM), lambda t, te, tv: (t, 0)),
        ),
        out_shape=jax.ShapeDtypeStruct((NT * T, DIM), jnp.float32),
    )(tile_expert, tile_valid, xf, srow,
      W1, W1, jnp.swapaxes(W2, 1, 2), jnp.swapaxes(W2, 1, 2), W3, W3)

    TT = 256
    z = pl.pallas_call(
        _shared_body,
        grid=(N // TT,),
        in_specs=[
            pl.BlockSpec((TT, DIM), lambda t: (t, 0)),
            pl.BlockSpec((NSHARE * INTER, DIM), lambda t: (0, 0)),
            pl.BlockSpec((DIM, NSHARE * INTER), lambda t: (0, 0)),
            pl.BlockSpec((NSHARE * INTER, DIM), lambda t: (0, 0)),
        ],
        out_specs=pl.BlockSpec((TT, DIM), lambda t: (t, 0)),
        out_shape=jax.ShapeDtypeStruct((N, DIM), jnp.float32),
    )(xf, Ws1, Ws2, Ws3)

    y = pl.pallas_call(
        _scatter_body,
        grid_spec=pltpu.PrefetchScalarGridSpec(
            num_scalar_prefetch=1,
            grid=(NC,),
            in_specs=[
                pl.BlockSpec((N, 8), lambda c, cv: (0, 0)),
                pl.BlockSpec((SC, DIM), lambda c, cv: (c, 0)),
                pl.BlockSpec((N, DIM), lambda c, cv: (0, 0)),
            ],
            out_specs=pl.BlockSpec((N, DIM), lambda c, cv: (0, 0)),
        ),
        out_shape=jax.ShapeDtypeStruct((N, DIM), jnp.float32),
    )(chunk_valid, pairs, og, z)

    return (y.reshape(b, n, d), aux_loss)


# speedup vs baseline: 1.0376x; 1.0376x over previous
import functools

import jax
import jax.numpy as jnp
from jax.experimental import pallas as pl
from jax.experimental.pallas import tpu as pltpu

DIM = 512
INTER = 1344
TOPK = 2
E = 64
NSHARE = 2
ROUTE_SCALE = 1.0
ALPHA = 0.1
N = 2048

T = 64
NT = 128


def _silu(v):
    return v * jax.nn.sigmoid(v)


def _router_body(x_ref, wg_ref, pairs_ref, cnt_ref, aux_ref):
    x = x_ref[...]
    wg = wg_ref[...]
    logits = jax.lax.dot_general(
        x, wg, (((1,), (1,)), ((), ())), preferred_element_type=jnp.float32)
    m = jnp.max(logits, axis=1, keepdims=True)
    ex = jnp.exp(logits - m)
    score = ex / jnp.sum(ex, axis=1, keepdims=True)

    iota = jax.lax.broadcasted_iota(jnp.int32, (N, E), 1)
    m1 = jnp.max(score, axis=1, keepdims=True)
    i1 = jnp.min(jnp.where(score == m1, iota, E), axis=1, keepdims=True)
    sc2 = jnp.where(iota == i1, -1.0, score)
    m2 = jnp.max(sc2, axis=1, keepdims=True)
    i2 = jnp.min(jnp.where(sc2 == m2, iota, E), axis=1, keepdims=True)

    oh1 = (iota == i1).astype(jnp.float32)
    oh2 = (iota == i2).astype(jnp.float32)
    oh12 = oh1 + oh2

    cnt = jnp.sum(oh12, axis=0, keepdims=True)
    fi = cnt / (TOPK * N / E)
    pi = jnp.mean(score, axis=0, keepdims=True)
    aux = jnp.sum(fi * pi) * ALPHA
    aux_ref[...] = jnp.full((8, 128), aux, jnp.float32)

    ir = jax.lax.broadcasted_iota(jnp.int32, (N, N), 0)
    ic = jax.lax.broadcasted_iota(jnp.int32, (N, N), 1)
    ltri = (ic < ir).astype(jnp.float32)
    cum = jax.lax.dot_general(ltri, oh12, (((1,), (0,)), ((), ())),
                              preferred_element_type=jnp.float32)
    rank1 = jnp.sum(cum * oh1, axis=1, keepdims=True)
    rank2 = jnp.sum(cum * oh2, axis=1, keepdims=True)

    tiles_per = jnp.floor((cnt + (T - 1)) * (1.0 / T))
    er = jax.lax.broadcasted_iota(jnp.int32, (E, E), 0)
    ec = jax.lax.broadcasted_iota(jnp.int32, (E, E), 1)
    etri = (er < ec).astype(jnp.float32)
    tile_start = jax.lax.dot_general(
        tiles_per, etri, (((1,), (0,)), ((), ())),
        preferred_element_type=jnp.float32)
    base1 = jnp.sum(tile_start * oh1, axis=1, keepdims=True) * T
    base2 = jnp.sum(tile_start * oh2, axis=1, keepdims=True) * T
    s1 = base1 + rank1
    s2 = base2 + rank2

    zf = jnp.zeros((N, 4), jnp.float32)
    pairs_ref[...] = jnp.concatenate([s1, s2, m1, m2, zf], axis=1)
    cnt_ref[...] = jnp.concatenate(
        [jnp.concatenate([cnt, jnp.zeros((1, 128 - E), jnp.float32)], axis=1),
         jnp.zeros((7, 128), jnp.float32)], axis=0)


def _grouped_body(te_ref, tv_ref, x_ref, srow_ref, pairs_ref, z_ref,
                  w1a_ref, w1b_ref, w2a_ref, w2b_ref, w3a_ref, w3b_ref,
                  y_ref):
    t = pl.program_id(0)

    @pl.when(t == 0)
    def _init():
        y_ref[...] = z_ref[...]

    @pl.when(tv_ref[t] == 1)
    def _live():
        s1r = srow_ref[0:1, :].astype(jnp.int32)
        s2r = srow_ref[1:2, :].astype(jnp.int32)
        jglob = jax.lax.broadcasted_iota(jnp.int32, (T, N), 0) + t * T
        onehot_g = ((jglob == s1r) | (jglob == s2r)).astype(jnp.float32)
        xs = jax.lax.dot_general(
            onehot_g, x_ref[...], (((1,), (0,)), ((), ())),
            preferred_element_type=jnp.float32)

        def half(w1_ref, w3_ref, w2_ref):
            w1 = w1_ref[0]
            w3 = w3_ref[0]
            w2t = w2_ref[0]
            h1 = jax.lax.dot_general(xs, w1, (((1,), (1,)), ((), ())),
                                     preferred_element_type=jnp.float32)
            h3 = jax.lax.dot_general(xs, w3, (((1,), (1,)), ((), ())),
                                     preferred_element_type=jnp.float32)
            h = _silu(h1) * h3
            return jax.lax.dot_general(
                h, w2t, (((1,), (0,)), ((), ())),
                preferred_element_type=jnp.float32)

        o = (half(w1a_ref, w3a_ref, w2a_ref)
             + half(w1b_ref, w3b_ref, w2b_ref))

        s1c = pairs_ref[:, 0:1].astype(jnp.int32)
        s2c = pairs_ref[:, 1:2].astype(jnp.int32)
        g1c = pairs_ref[:, 2:3]
        g2c = pairs_ref[:, 3:4]
        jg2 = jax.lax.broadcasted_iota(jnp.int32, (N, T), 1) + t * T
        scat = ((jg2 == s1c).astype(jnp.float32) * g1c
                + (jg2 == s2c).astype(jnp.float32) * g2c)
        y_ref[...] += jax.lax.dot_general(
            scat, o, (((1,), (0,)), ((), ())),
            preferred_element_type=jnp.float32)


def _shared_body(x_ref, w1_ref, w2_ref, w3_ref, o_ref):
    xs = x_ref[...]
    h1 = jax.lax.dot_general(xs, w1_ref[...], (((1,), (1,)), ((), ())),
                             preferred_element_type=jnp.float32)
    h3 = jax.lax.dot_general(xs, w3_ref[...], (((1,), (1,)), ((), ())),
                             preferred_element_type=jnp.float32)
    h = _silu(h1) * h3
    o_ref[...] = jax.lax.dot_general(h, w2_ref[...], (((1,), (1,)), ((), ())),
                                     preferred_element_type=jnp.float32)


@functools.partial(jax.jit, static_argnames=())
def kernel(x, Wg, W1, W2, W3, Ws1, Ws2, Ws3):
    b, n, d = x.shape
    xf = x.reshape(n, d)

    pairs, cnt8, aux8 = pl.pallas_call(
        _router_body,
        out_shape=[
            jax.ShapeDtypeStruct((N, 8), jnp.float32),
            jax.ShapeDtypeStruct((8, 128), jnp.float32),
            jax.ShapeDtypeStruct((8, 128), jnp.float32),
        ],
    )(xf, Wg)
    aux_loss = aux8[0, 0]

    counts = cnt8[0, :E].astype(jnp.int32)
    tiles_per = (counts + T - 1) // T
    bounds = jnp.cumsum(tiles_per)
    total_tiles = bounds[-1]
    tile_expert = jnp.minimum(
        jnp.searchsorted(bounds, jnp.arange(NT, dtype=jnp.int32),
                         side='right'),
        E - 1).astype(jnp.int32)
    tile_valid = (jnp.arange(NT, dtype=jnp.int32)
                  < total_tiles).astype(jnp.int32)
    srow = pairs.T

    TT = 256
    z = pl.pallas_call(
        _shared_body,
        grid=(N // TT,),
        in_specs=[
            pl.BlockSpec((TT, DIM), lambda t: (t, 0)),
            pl.BlockSpec((NSHARE * INTER, DIM), lambda t: (0, 0)),
            pl.BlockSpec((DIM, NSHARE * INTER), lambda t: (0, 0)),
            pl.BlockSpec((NSHARE * INTER, DIM), lambda t: (0, 0)),
        ],
        out_specs=pl.BlockSpec((TT, DIM), lambda t: (t, 0)),
        out_shape=jax.ShapeDtypeStruct((N, DIM), jnp.float32),
    )(xf, Ws1, Ws2, Ws3)

    y = pl.pallas_call(
        _grouped_body,
        grid_spec=pltpu.PrefetchScalarGridSpec(
            num_scalar_prefetch=2,
            grid=(NT,),
            in_specs=[
                pl.BlockSpec((N, DIM), lambda t, te, tv: (0, 0)),
                pl.BlockSpec((8, N), lambda t, te, tv: (0, 0)),
                pl.BlockSpec((N, 8), lambda t, te, tv: (0, 0)),
                pl.BlockSpec((N, DIM), lambda t, te, tv: (0, 0)),
                pl.BlockSpec((1, INTER // 2, DIM),
                             lambda t, te, tv: (te[t], 0, 0)),
                pl.BlockSpec((1, INTER // 2, DIM),
                             lambda t, te, tv: (te[t], 1, 0)),
                pl.BlockSpec((1, INTER // 2, DIM),
                             lambda t, te, tv: (te[t], 0, 0)),
                pl.BlockSpec((1, INTER // 2, DIM),
                             lambda t, te, tv: (te[t], 1, 0)),
                pl.BlockSpec((1, INTER // 2, DIM),
                             lambda t, te, tv: (te[t], 0, 0)),
                pl.BlockSpec((1, INTER // 2, DIM),
                             lambda t, te, tv: (te[t], 1, 0)),
            ],
            out_specs=pl.BlockSpec((N, DIM), lambda t, te, tv: (0, 0)),
        ),
        out_shape=jax.ShapeDtypeStruct((N, DIM), jnp.float32),
    )(tile_expert, tile_valid, xf, srow, pairs, z,
      W1, W1, jnp.swapaxes(W2, 1, 2), jnp.swapaxes(W2, 1, 2), W3, W3)

    return (y.reshape(b, n, d), aux_loss)
